# Initial kernel scaffold; baseline (speedup 1.0000x reference)
import jax
import jax.numpy as jnp
from jax.experimental import pallas as pl


def kernel(x, edge_index, W_l, W_r, W1, b1, W2, b2):
    raise NotImplementedError("write your pallas kernel here")


# speedup vs baseline: 199.3252x; 199.3252x over previous
import functools

import jax
import jax.numpy as jnp
from jax import lax
from jax.experimental import pallas as pl
from jax.experimental.pallas import tpu as pltpu
from jax.experimental.pallas import tpu_sc as plsc

_N = 99999
_K = _N // 3
_NC = 2
_NS = 16
_NW = _NC * _NS
_CHUNK = 2048
_ACC_P = 100608
_STRIPE = _ACC_P // _NS
_KP = _ACC_P // 3
_QS = _KP // _NS


def _make_sc_kernel(nch):
    mesh = plsc.VectorSubcoreMesh(core_axis_name="c", subcore_axis_name="s",
                                  num_cores=_NC, num_subcores=_NS)

    @functools.partial(
        pl.kernel,
        out_type=jax.ShapeDtypeStruct((3 * _KP,), jnp.float32),
        mesh=mesh,
        compiler_params=pltpu.CompilerParams(
            needs_layout_passes=False, use_tc_tiling_on_sc=False),
        scratch_types=[
            pltpu.VMEM((_ACC_P,), jnp.float32),
            pltpu.VMEM((2, _CHUNK), jnp.int32),
            pltpu.VMEM((2, 16, 128), jnp.int32),
            pltpu.VMEM((16, 128), jnp.float32),
            pltpu.VMEM((_STRIPE,), jnp.float32),
            pltpu.VMEM((_QS,), jnp.float32),
            pltpu.VMEM_SHARED((_ACC_P,), jnp.float32),
            pltpu.SemaphoreType.DMA,
            pltpu.SemaphoreType.DMA,
            pltpu.SemaphoreType.DMA,
            pltpu.SemaphoreType.DMA,
        ],
    )
    def sc_kernel(x_hbm, srcr_hbm, dstr_hbm, zeros_hbm, q_hbm,
                  x_v, src_v, dst_v, vals_v, cbuf, qbuf, acc_sh,
                  sem_s0, sem_s1, sem_d0, sem_d1):
        cid = lax.axis_index("c")
        sid = lax.axis_index("s")
        wid = sid * _NC + cid
        sem_s = (sem_s0, sem_s1)
        sem_d = (sem_d0, sem_d1)

        pltpu.sync_copy(x_hbm, x_v)
        pltpu.sync_copy(zeros_hbm, acc_sh.at[pl.ds(sid * _STRIPE, _STRIPE)])
        plsc.subcore_barrier()

        row0 = wid * nch

        def start(g, b):
            pltpu.async_copy(srcr_hbm.at[row0 + g], src_v.at[b], sem_s[b])
            pltpu.async_copy(dstr_hbm.at[row0 + g], dst_v.at[b], sem_d[b])

        def wait(g, b):
            pltpu.make_async_copy(srcr_hbm.at[row0 + g], src_v.at[b],
                                  sem_s[b]).wait()
            pltpu.make_async_copy(dstr_hbm.at[row0 + g], dst_v.at[b],
                                  sem_d[b]).wait()

        def process(b):
            for j in range(16):
                for c in range(8):
                    s = src_v[b, pl.ds(128 * j + 16 * c, 16)]
                    v = plsc.load_gather(x_v, [s])
                    vals_v[j, pl.ds(16 * c, 16)] = v
            for j in range(16):
                pltpu.sync_copy(vals_v.at[j], acc_sh.at[dst_v.at[b, j]],
                                add=True)

        start(0, 0)
        start(1, 1)

        def outer(i, carry):
            for b in range(2):
                g = 2 * i + b
                wait(g, b)
                process(b)

                @pl.when(g + 2 < nch)
                def _():
                    start(g + 2, b)
            return carry

        lax.fori_loop(0, nch // 2, outer, 0)
        plsc.subcore_barrier()

        t = sid
        pltpu.sync_copy(acc_sh.at[pl.ds(t * _STRIPE, _STRIPE)], cbuf)
        lanes = jnp.arange(16, dtype=jnp.int32)
        for v in range(_QS // 16):
            idx = 3 * (16 * v) + 3 * lanes
            qbuf[pl.ds(16 * v, 16)] = plsc.load_gather(cbuf, [idx])
        pltpu.sync_copy(qbuf, q_hbm.at[pl.ds(cid * _KP + t * _QS, _QS)])

        @pl.when(cid == 0)
        def _():
            base0 = t * _QS
            for v in range(_QS // 16):
                idx = 3 * (base0 + 16 * v) + 3 * lanes
                qbuf[pl.ds(16 * v, 16)] = plsc.load_gather(x_v, [idx])
            pltpu.sync_copy(qbuf, q_hbm.at[pl.ds(2 * _KP + t * _QS, _QS)])

    return sc_kernel


def _combine_body(p_ref, q0_ref, q1_ref, q2_ref, o_ref):
    h = p_ref[0] * (q0_ref[...] + q1_ref[...]) + p_ref[1] * q2_ref[...]
    a0 = jnp.maximum(p_ref[2] * h + p_ref[3], 0.0)
    a1 = jnp.maximum(p_ref[4] * h + p_ref[5], 0.0)
    o_ref[...] = p_ref[6] * a0 + p_ref[7] * a1 + p_ref[8]


def kernel(x, edge_index, W_l, W_r, W1, b1, W2, b2):
    x1 = x.reshape(-1)
    src = edge_index[0]
    dst = edge_index[1]
    e = src.shape[0]

    per_worker = _NW * _CHUNK
    nch = -(-e // per_worker)
    if nch % 2:
        nch += 1
    ep = nch * per_worker
    pad = ep - e

    src_p = jnp.concatenate([src, jnp.zeros((pad,), jnp.int32)])
    dst_p = jnp.concatenate([dst, jnp.ones((pad,), jnp.int32)])
    srcr = src_p.reshape(ep // _CHUNK, _CHUNK)
    dstr = dst_p.reshape(ep // _CHUNK, 16, 128)
    x_pad = jnp.concatenate(
        [x1, jnp.zeros((_ACC_P - _N,), jnp.float32)])
    zeros = jnp.zeros((_STRIPE,), jnp.float32)

    q = _make_sc_kernel(nch)(x_pad, srcr, dstr, zeros)

    params = jnp.stack([
        W_l[0, 0], W_r[0, 0],
        W1[0, 0], b1[0], W1[1, 0], b1[1],
        W2[0, 0], W2[0, 1], b2[0],
    ])
    q2d = q.reshape(3, _KP // 128, 128)
    out2d = pl.pallas_call(
        _combine_body,
        out_shape=jax.ShapeDtypeStruct((_KP // 128, 128), jnp.float32),
        in_specs=[
            pl.BlockSpec(memory_space=pltpu.SMEM),
            pl.BlockSpec(memory_space=pltpu.VMEM),
            pl.BlockSpec(memory_space=pltpu.VMEM),
            pl.BlockSpec(memory_space=pltpu.VMEM),
        ],
    )(params, q2d[0], q2d[1], q2d[2])
    return out2d.reshape(-1)[:_K]


# speedup vs baseline: 215.9251x; 1.0833x over previous
import functools

import jax
import jax.numpy as jnp
from jax import lax
from jax.experimental import pallas as pl
from jax.experimental.pallas import tpu as pltpu
from jax.experimental.pallas import tpu_sc as plsc

_N = 99999
_K = _N // 3
_NC = 2
_NS = 16
_NW = _NC * _NS
_CHUNK = 1536
_CR = _CHUNK // 128
_XT = 100000
_ACC_P = 100608
_STRIPE = _ACC_P // _NS
_KP = _ACC_P // 3
_QS = _KP // _NS


def _make_sc_kernel(nch):
    mesh = plsc.VectorSubcoreMesh(core_axis_name="c", subcore_axis_name="s",
                                  num_cores=_NC, num_subcores=_NS)

    @functools.partial(
        pl.kernel,
        out_type=jax.ShapeDtypeStruct((3 * _KP,), jnp.float32),
        mesh=mesh,
        compiler_params=pltpu.CompilerParams(
            needs_layout_passes=False, use_tc_tiling_on_sc=False),
        scratch_types=[
            pltpu.VMEM((_XT,), jnp.float32),
            pltpu.VMEM((3, _CHUNK), jnp.int32),
            pltpu.VMEM((3, _CR, 128), jnp.int32),
            pltpu.VMEM((3, _CR, 128), jnp.float32),
            pltpu.VMEM((_STRIPE,), jnp.float32),
            pltpu.VMEM((_QS,), jnp.float32),
            pltpu.VMEM_SHARED((_ACC_P,), jnp.float32),
            pltpu.SemaphoreType.DMA,
            pltpu.SemaphoreType.DMA,
            pltpu.SemaphoreType.DMA,
            pltpu.SemaphoreType.DMA,
            pltpu.SemaphoreType.DMA,
            pltpu.SemaphoreType.DMA,
            pltpu.SemaphoreType.DMA,
            pltpu.SemaphoreType.DMA,
            pltpu.SemaphoreType.DMA,
        ],
    )
    def sc_kernel(x_hbm, srcr_hbm, dstr_hbm, zeros_hbm, q_hbm,
                  x_v, src_v, dst_v, vals_v, cbuf, qbuf, acc_sh,
                  sem_s0, sem_s1, sem_s2, sem_d0, sem_d1, sem_d2,
                  sem_a0, sem_a1, sem_a2):
        cid = lax.axis_index("c")
        sid = lax.axis_index("s")
        wid = sid * _NC + cid
        sem_s = (sem_s0, sem_s1, sem_s2)
        sem_d = (sem_d0, sem_d1, sem_d2)
        sem_a = (sem_a0, sem_a1, sem_a2)

        pltpu.sync_copy(x_hbm, x_v)
        pltpu.sync_copy(zeros_hbm, acc_sh.at[pl.ds(sid * _STRIPE, _STRIPE)])
        plsc.subcore_barrier()

        row0 = wid * nch

        def start(g, b):
            pltpu.async_copy(srcr_hbm.at[row0 + g], src_v.at[b], sem_s[b])
            pltpu.async_copy(dstr_hbm.at[row0 + g], dst_v.at[b], sem_d[b])

        def wait(g, b):
            pltpu.make_async_copy(srcr_hbm.at[row0 + g], src_v.at[b],
                                  sem_s[b]).wait()
            pltpu.make_async_copy(dstr_hbm.at[row0 + g], dst_v.at[b],
                                  sem_d[b]).wait()

        def gather(b):
            for j in range(_CR):
                for c in range(8):
                    s = src_v[b, pl.ds(128 * j + 16 * c, 16)]
                    v = plsc.load_gather(x_v, [s])
                    vals_v[b, j, pl.ds(16 * c, 16)] = v

        def fire_scatter(b):
            for j in range(_CR):
                pltpu.async_copy(vals_v.at[b, j], acc_sh.at[dst_v.at[b, j]],
                                 sem_a[b], add=True)

        def drain_scatter(b):
            for j in range(_CR):
                pltpu.make_async_copy(vals_v.at[b, j],
                                      acc_sh.at[dst_v.at[b, j]],
                                      sem_a[b]).wait()

        start(0, 0)
        start(1, 1)

        def outer(i, carry):
            for b in range(3):
                g = 3 * i + b
                bprev = (b + 2) % 3

                @pl.when(g >= 1)
                def _():
                    drain_scatter(bprev)

                @pl.when(g + 2 < nch)
                def _():
                    start(g + 2, bprev)

                wait(g, b)
                gather(b)
                fire_scatter(b)
            return carry

        lax.fori_loop(0, nch // 3, outer, 0)
        drain_scatter((nch - 1) % 3)
        plsc.subcore_barrier()

        t = sid
        pltpu.sync_copy(acc_sh.at[pl.ds(t * _STRIPE, _STRIPE)], cbuf)
        lanes = jnp.arange(16, dtype=jnp.int32)
        for v in range(_QS // 16):
            idx = 3 * (16 * v) + 3 * lanes
            qbuf[pl.ds(16 * v, 16)] = plsc.load_gather(cbuf, [idx])
        pltpu.sync_copy(qbuf, q_hbm.at[pl.ds(cid * _KP + t * _QS, _QS)])

        @pl.when(cid == 0)
        def _():
            base0 = t * _QS
            for v in range(_QS // 16):
                idx = 3 * (base0 + 16 * v) + 3 * lanes
                idx = jnp.minimum(idx, _N)
                qbuf[pl.ds(16 * v, 16)] = plsc.load_gather(x_v, [idx])
            pltpu.sync_copy(qbuf, q_hbm.at[pl.ds(2 * _KP + t * _QS, _QS)])

    return sc_kernel


def _combine_body(p_ref, q0_ref, q1_ref, q2_ref, o_ref):
    h = p_ref[0] * (q0_ref[...] + q1_ref[...]) + p_ref[1] * q2_ref[...]
    a0 = jnp.maximum(p_ref[2] * h + p_ref[3], 0.0)
    a1 = jnp.maximum(p_ref[4] * h + p_ref[5], 0.0)
    o_ref[...] = p_ref[6] * a0 + p_ref[7] * a1 + p_ref[8]


def kernel(x, edge_index, W_l, W_r, W1, b1, W2, b2):
    x1 = x.reshape(-1)
    src = edge_index[0]
    dst = edge_index[1]
    e = src.shape[0]

    per_worker = _NW * _CHUNK
    nch = -(-e // per_worker)
    nch = -(-nch // 3) * 3
    ep = nch * per_worker
    pad = ep - e

    src_p = jnp.concatenate([src, jnp.zeros((pad,), jnp.int32)])
    dst_p = jnp.concatenate([dst, jnp.ones((pad,), jnp.int32)])
    srcr = src_p.reshape(ep // _CHUNK, _CHUNK)
    dstr = dst_p.reshape(ep // _CHUNK, _CR, 128)
    x_pad = jnp.concatenate(
        [x1, jnp.zeros((_XT - _N,), jnp.float32)])
    zeros = jnp.zeros((_STRIPE,), jnp.float32)

    q = _make_sc_kernel(nch)(x_pad, srcr, dstr, zeros)

    params = jnp.stack([
        W_l[0, 0], W_r[0, 0],
        W1[0, 0], b1[0], W1[1, 0], b1[1],
        W2[0, 0], W2[0, 1], b2[0],
    ])
    q2d = q.reshape(3, _KP // 128, 128)
    out2d = pl.pallas_call(
        _combine_body,
        out_shape=jax.ShapeDtypeStruct((_KP // 128, 128), jnp.float32),
        in_specs=[
            pl.BlockSpec(memory_space=pltpu.SMEM),
            pl.BlockSpec(memory_space=pltpu.VMEM),
            pl.BlockSpec(memory_space=pltpu.VMEM),
            pl.BlockSpec(memory_space=pltpu.VMEM),
        ],
    )(params, q2d[0], q2d[1], q2d[2])
    return out2d.reshape(-1)[:_K]
